# baseline (speedup 1.0000x reference)
import functools

import jax
import jax.numpy as jnp
from jax import lax
from jax.experimental import pallas as pl
from jax.experimental.pallas import tpu as pltpu
from jax.experimental.pallas import tpu_sc as plsc

_NR = 8
_UNROLL = 2
_B_SC = 8
_NW = 32
_GP = 104


def _tc_detection_kernel(rois_ref, ids_ref, gt_ref, out_ref):
    r = rois_ref[0]
    y1 = r[0]
    x1 = r[1]
    y2 = r[2]
    x2 = r[3]
    a1 = (y2 - y1) * (x2 - x1)
    G = gt_ref.shape[1]

    def gbody(g, carry):
        nc, cb = carry
        gy1 = gt_ref[0, g, 0]
        gx1 = gt_ref[0, g, 1]
        gy2 = gt_ref[0, g, 2]
        gx2 = gt_ref[0, g, 3]
        gid = ids_ref[0, g, 0]
        valid = ((jnp.abs(gy1) > 0) | (jnp.abs(gx1) > 0) |
                 (jnp.abs(gy2) > 0) | (jnp.abs(gx2) > 0))
        neg1 = jnp.float32(-1.0)
        is_nc = valid & (gid > 0)
        is_c = valid & (gid < 0)

        a2 = (gy2 - gy1) * (gx2 - gx1)
        iy1 = jnp.maximum(y1, gy1)
        ix1 = jnp.maximum(x1, gx1)
        iy2 = jnp.minimum(y2, gy2)
        ix2 = jnp.minimum(x2, gx2)
        inter = jnp.maximum(iy2 - iy1, 0.0) * jnp.maximum(ix2 - ix1, 0.0)
        union = a1 + a2 - inter
        iou = inter / jnp.maximum(union, 1e-8)
        nc = jnp.maximum(nc, jnp.where(is_nc, iou, neg1))
        cb = jnp.maximum(cb, jnp.where(is_c, iou, neg1))
        return nc, cb

    init = jnp.full_like(a1, -1.0)
    nc_max, c_max = jax.lax.fori_loop(0, G, gbody, (init, init),
                                      unroll=_UNROLL)

    roi_valid = ((jnp.abs(y1) > 0) | (jnp.abs(x1) > 0) |
                 (jnp.abs(y2) > 0) | (jnp.abs(x2) > 0))
    neg_one = jnp.float32(-1.0)
    nc_max = jnp.where(roi_valid, nc_max, neg_one)
    c_max = jnp.where(roi_valid, c_max, neg_one)
    pos = ((nc_max >= 0.5) & roi_valid).astype(jnp.float32)
    neg = ((nc_max < 0.5) & (c_max < 0.001) & roi_valid).astype(jnp.float32)
    out_ref[0, 0] = nc_max
    out_ref[0, 1] = c_max
    out_ref[0, 2] = pos
    out_ref[0, 3] = neg


def _tc_detection(rois_t, gt_ids, gt_boxes):
    B, _, N = rois_t.shape
    G = gt_boxes.shape[1]
    NL = N // _NR
    rb = _NR // 8
    rois_r = rois_t.reshape(B, 4, _NR, NL)
    out = pl.pallas_call(
        _tc_detection_kernel,
        grid=(B, rb),
        in_specs=[
            pl.BlockSpec((1, 4, 8, NL), lambda b, r: (b, 0, r, 0)),
            pl.BlockSpec((1, G, 1), lambda b, r: (b, 0, 0),
                         memory_space=pltpu.SMEM),
            pl.BlockSpec((1, G, 4), lambda b, r: (b, 0, 0),
                         memory_space=pltpu.SMEM),
        ],
        out_specs=pl.BlockSpec((1, 4, 8, NL), lambda b, r: (b, 0, r, 0)),
        out_shape=jax.ShapeDtypeStruct((B, 4, _NR, NL), jnp.float32),
        compiler_params=pltpu.CompilerParams(
            dimension_semantics=("parallel", "parallel"),
        ),
    )(rois_r, gt_ids.reshape(B, G, 1), gt_boxes)
    return out.reshape(B, 4, N)


def _sc_detection(rois_tp, gt_ids_b, gt_boxes_b):
    Bs, _, NP = rois_tp.shape
    GP = gt_ids_b.shape[1]
    SLICE = NP // _NW
    mesh = plsc.VectorSubcoreMesh(core_axis_name="c", subcore_axis_name="s")

    @functools.partial(
        pl.kernel, mesh=mesh,
        out_type=jax.ShapeDtypeStruct((Bs, 4, NP), jnp.float32),
        scratch_types=[
            pltpu.VMEM((4, SLICE), jnp.float32),
            pltpu.VMEM((4, SLICE), jnp.float32),
            pltpu.VMEM((4, GP, 16), jnp.float32),
            pltpu.VMEM((GP, 16), jnp.int32),
        ],
    )
    def k(rois_hbm, ids_hbm, gt_hbm, out_hbm, coords_v, outb_v, gt_v, ids_v):
        wid = lax.axis_index("s") * 2 + lax.axis_index("c")
        base = wid * SLICE
        neg1 = jnp.full((16,), -1.0, jnp.float32)
        zero = jnp.zeros((16,), jnp.float32)
        izero = jnp.zeros((16,), jnp.int32)
        eps = jnp.full((16,), 1e-8, jnp.float32)
        half = jnp.full((16,), 0.5, jnp.float32)
        milli = jnp.full((16,), 0.001, jnp.float32)
        one = jnp.full((16,), 1.0, jnp.float32)
        for b in range(Bs):
            pltpu.sync_copy(gt_hbm.at[b], gt_v)
            pltpu.sync_copy(ids_hbm.at[b], ids_v)
            pltpu.sync_copy(rois_hbm.at[b, :, pl.ds(base, SLICE)], coords_v)

            def group_body(j, _):
                off = j * 32
                ys, xs, y2s, x2s, a1s = [], [], [], [], []
                for t in range(2):
                    o = off + t * 16
                    y1 = coords_v[0, pl.ds(o, 16)]
                    x1 = coords_v[1, pl.ds(o, 16)]
                    y2 = coords_v[2, pl.ds(o, 16)]
                    x2 = coords_v[3, pl.ds(o, 16)]
                    ys.append(y1)
                    xs.append(x1)
                    y2s.append(y2)
                    x2s.append(x2)
                    a1s.append((y2 - y1) * (x2 - x1))
                init = jnp.full((16,), -1.0, jnp.float32)

                def gt_body(g, carry):
                    accs = list(carry)
                    gy1 = gt_v[0, g]
                    gx1 = gt_v[1, g]
                    gy2 = gt_v[2, g]
                    gx2 = gt_v[3, g]
                    gid = ids_v[g]
                    valid = ((gy1 != zero) | (gx1 != zero) |
                             (gy2 != zero) | (gx2 != zero))
                    is_nc = valid & (gid > izero)
                    is_c = valid & (gid < izero)
                    a2 = (gy2 - gy1) * (gx2 - gx1)
                    for t in range(2):
                        iy1 = jnp.maximum(ys[t], gy1)
                        ix1 = jnp.maximum(xs[t], gx1)
                        iy2 = jnp.minimum(y2s[t], gy2)
                        ix2 = jnp.minimum(x2s[t], gx2)
                        inter = (jnp.maximum(iy2 - iy1, zero) *
                                 jnp.maximum(ix2 - ix1, zero))
                        union = a1s[t] + a2 - inter
                        iou = inter / jnp.maximum(union, eps)
                        accs[2 * t] = jnp.maximum(
                            accs[2 * t], jnp.where(is_nc, iou, neg1))
                        accs[2 * t + 1] = jnp.maximum(
                            accs[2 * t + 1], jnp.where(is_c, iou, neg1))
                    return tuple(accs)

                accs = lax.fori_loop(0, GP, gt_body, (init,) * 4)
                for t in range(2):
                    o = off + t * 16
                    nc, cb = accs[2 * t], accs[2 * t + 1]
                    rv = ((ys[t] != zero) | (xs[t] != zero) |
                          (y2s[t] != zero) | (x2s[t] != zero))
                    nc = jnp.where(rv, nc, neg1)
                    cb = jnp.where(rv, cb, neg1)
                    pos = jnp.where((nc >= half) & rv, one, zero)
                    neg = jnp.where((nc < half) & (cb < milli) & rv, one, zero)
                    outb_v[0, pl.ds(o, 16)] = nc
                    outb_v[1, pl.ds(o, 16)] = cb
                    outb_v[2, pl.ds(o, 16)] = pos
                    outb_v[3, pl.ds(o, 16)] = neg
                return 0

            lax.fori_loop(0, SLICE // 32, group_body, 0)
            pltpu.sync_copy(outb_v, out_hbm.at[b, :, pl.ds(base, SLICE)])

    return k(rois_tp, gt_ids_b, gt_boxes_b)


def kernel(rois, gt_ids, gt_boxes):
    B, N, _ = rois.shape
    G = gt_boxes.shape[1]
    rois_t = jnp.transpose(rois, (0, 2, 1))
    bs = _B_SC
    parts = []
    if bs > 0:
        NP = -(-N // (16 * _NW)) * (16 * _NW)
        rois_tp = jnp.pad(rois_t[:bs], ((0, 0), (0, 0), (0, NP - N)))
        ids_p = jnp.pad(gt_ids[:bs], ((0, 0), (0, _GP - G)))
        gts_p = jnp.pad(gt_boxes[:bs], ((0, 0), (0, _GP - G), (0, 0)))
        ids_b = jnp.broadcast_to(ids_p[:, :, None], (bs, _GP, 16))
        gts_b = jnp.broadcast_to(
            jnp.transpose(gts_p, (0, 2, 1))[:, :, :, None], (bs, 4, _GP, 16))
        parts.append(_sc_detection(rois_tp, ids_b, gts_b)[:, :, :N])
    if bs < B:
        parts.append(_tc_detection(rois_t[bs:], gt_ids[bs:], gt_boxes[bs:]))
    if len(parts) == 1:
        return parts[0]
    return jnp.concatenate(parts, axis=0)


# speedup vs baseline: 1.1808x; 1.1808x over previous
import functools

import jax
import jax.numpy as jnp
from jax import lax
from jax.experimental import pallas as pl
from jax.experimental.pallas import tpu as pltpu
from jax.experimental.pallas import tpu_sc as plsc

_NR = 8
_UNROLL = 2
_B_SC = 8
_NW = 32
_GP = 104


def _tc_detection_kernel(rois_ref, ids_ref, gt_ref, out_ref):
    r = rois_ref[0]
    y1 = r[0]
    x1 = r[1]
    y2 = r[2]
    x2 = r[3]
    a1 = (y2 - y1) * (x2 - x1)
    G = gt_ref.shape[1]

    def gbody(g, carry):
        nc, cb = carry
        gy1 = gt_ref[0, g, 0]
        gx1 = gt_ref[0, g, 1]
        gy2 = gt_ref[0, g, 2]
        gx2 = gt_ref[0, g, 3]
        gid = ids_ref[0, g, 0]
        valid = ((jnp.abs(gy1) > 0) | (jnp.abs(gx1) > 0) |
                 (jnp.abs(gy2) > 0) | (jnp.abs(gx2) > 0))
        neg1 = jnp.float32(-1.0)
        is_nc = valid & (gid > 0)
        is_c = valid & (gid < 0)

        a2 = (gy2 - gy1) * (gx2 - gx1)
        iy1 = jnp.maximum(y1, gy1)
        ix1 = jnp.maximum(x1, gx1)
        iy2 = jnp.minimum(y2, gy2)
        ix2 = jnp.minimum(x2, gx2)
        inter = jnp.maximum(iy2 - iy1, 0.0) * jnp.maximum(ix2 - ix1, 0.0)
        union = a1 + a2 - inter
        iou = inter / jnp.maximum(union, 1e-8)
        nc = jnp.maximum(nc, jnp.where(is_nc, iou, neg1))
        cb = jnp.maximum(cb, jnp.where(is_c, iou, neg1))
        return nc, cb

    init = jnp.full_like(a1, -1.0)
    nc_max, c_max = jax.lax.fori_loop(0, G, gbody, (init, init),
                                      unroll=_UNROLL)

    roi_valid = ((jnp.abs(y1) > 0) | (jnp.abs(x1) > 0) |
                 (jnp.abs(y2) > 0) | (jnp.abs(x2) > 0))
    neg_one = jnp.float32(-1.0)
    nc_max = jnp.where(roi_valid, nc_max, neg_one)
    c_max = jnp.where(roi_valid, c_max, neg_one)
    pos = ((nc_max >= 0.5) & roi_valid).astype(jnp.float32)
    neg = ((nc_max < 0.5) & (c_max < 0.001) & roi_valid).astype(jnp.float32)
    out_ref[0, 0] = nc_max
    out_ref[0, 1] = c_max
    out_ref[0, 2] = pos
    out_ref[0, 3] = neg


def _tc_detection(rois_t, gt_ids, gt_boxes):
    B, _, N = rois_t.shape
    G = gt_boxes.shape[1]
    NL = N // _NR
    rb = _NR // 8
    rois_r = rois_t.reshape(B, 4, _NR, NL)
    out = pl.pallas_call(
        _tc_detection_kernel,
        grid=(B, rb),
        in_specs=[
            pl.BlockSpec((1, 4, 8, NL), lambda b, r: (b, 0, r, 0)),
            pl.BlockSpec((1, G, 1), lambda b, r: (b, 0, 0),
                         memory_space=pltpu.SMEM),
            pl.BlockSpec((1, G, 4), lambda b, r: (b, 0, 0),
                         memory_space=pltpu.SMEM),
        ],
        out_specs=pl.BlockSpec((1, 4, 8, NL), lambda b, r: (b, 0, r, 0)),
        out_shape=jax.ShapeDtypeStruct((B, 4, _NR, NL), jnp.float32),
        compiler_params=pltpu.CompilerParams(
            dimension_semantics=("parallel", "parallel"),
        ),
    )(rois_r, gt_ids.reshape(B, G, 1), gt_boxes)
    return out.reshape(B, 4, N)


def _sc_detection(rois_tp, gt_ids_b, gt_boxes_b):
    Bs, _, NP = rois_tp.shape
    GP = gt_ids_b.shape[1]
    SLICE = NP // _NW
    mesh = plsc.VectorSubcoreMesh(core_axis_name="c", subcore_axis_name="s")

    @functools.partial(
        pl.kernel, mesh=mesh,
        out_type=jax.ShapeDtypeStruct((Bs, 4, NP), jnp.float32),
        scratch_types=[
            pltpu.VMEM((4, SLICE), jnp.float32),
            pltpu.VMEM((4, SLICE), jnp.float32),
            pltpu.VMEM((4, GP, 16), jnp.float32),
            pltpu.VMEM((GP, 16), jnp.int32),
            pltpu.VMEM((GP, 16), jnp.float32),
            pltpu.VMEM((GP, 16), jnp.float32),
            pltpu.VMEM((GP, 16), jnp.float32),
        ],
    )
    def k(rois_hbm, ids_hbm, gt_hbm, out_hbm, coords_v, outb_v, gt_v, ids_v,
          a2_v, cnc_v, cc_v):
        wid = lax.axis_index("s") * 2 + lax.axis_index("c")
        base = wid * SLICE
        neg1 = jnp.full((16,), -1.0, jnp.float32)
        zero = jnp.zeros((16,), jnp.float32)
        izero = jnp.zeros((16,), jnp.int32)
        eps = jnp.full((16,), 1e-8, jnp.float32)
        half = jnp.full((16,), 0.5, jnp.float32)
        milli = jnp.full((16,), 0.001, jnp.float32)
        one = jnp.full((16,), 1.0, jnp.float32)
        two = jnp.full((16,), 2.0, jnp.float32)
        K = 4
        for b in range(Bs):
            pltpu.sync_copy(gt_hbm.at[b], gt_v)
            pltpu.sync_copy(ids_hbm.at[b], ids_v)
            pltpu.sync_copy(rois_hbm.at[b, :, pl.ds(base, SLICE)], coords_v)

            def prep_body(g, _):
                gy1 = gt_v[0, g]
                gx1 = gt_v[1, g]
                gy2 = gt_v[2, g]
                gx2 = gt_v[3, g]
                gid = ids_v[g]
                valid = ((gy1 != zero) | (gx1 != zero) |
                         (gy2 != zero) | (gx2 != zero))
                a2_v[g] = (gy2 - gy1) * (gx2 - gx1)
                cnc_v[g] = jnp.where(valid & (gid > izero), two, neg1)
                cc_v[g] = jnp.where(valid & (gid < izero), two, neg1)
                return 0

            lax.fori_loop(0, GP, prep_body, 0)

            def group_body(j, _):
                off = j * 16 * K
                ys, xs, y2s, x2s, a1s = [], [], [], [], []
                for t in range(K):
                    o = off + t * 16
                    y1 = coords_v[0, pl.ds(o, 16)]
                    x1 = coords_v[1, pl.ds(o, 16)]
                    y2 = coords_v[2, pl.ds(o, 16)]
                    x2 = coords_v[3, pl.ds(o, 16)]
                    ys.append(y1)
                    xs.append(x1)
                    y2s.append(y2)
                    x2s.append(x2)
                    a1s.append((y2 - y1) * (x2 - x1))
                init = jnp.full((16,), -1.0, jnp.float32)

                def gt_body(g, carry):
                    accs = list(carry)
                    gy1 = gt_v[0, g]
                    gx1 = gt_v[1, g]
                    gy2 = gt_v[2, g]
                    gx2 = gt_v[3, g]
                    a2 = a2_v[g]
                    capnc = cnc_v[g]
                    capc = cc_v[g]
                    for t in range(K):
                        iy1 = jnp.maximum(ys[t], gy1)
                        ix1 = jnp.maximum(xs[t], gx1)
                        iy2 = jnp.minimum(y2s[t], gy2)
                        ix2 = jnp.minimum(x2s[t], gx2)
                        inter = (jnp.maximum(iy2 - iy1, zero) *
                                 jnp.maximum(ix2 - ix1, zero))
                        union = a1s[t] + a2 - inter
                        iou = inter / jnp.maximum(union, eps)
                        accs[2 * t] = jnp.maximum(
                            accs[2 * t], jnp.minimum(iou, capnc))
                        accs[2 * t + 1] = jnp.maximum(
                            accs[2 * t + 1], jnp.minimum(iou, capc))
                    return tuple(accs)

                accs = lax.fori_loop(0, GP, gt_body, (init,) * (2 * K),
                                     unroll=2)
                for t in range(K):
                    o = off + t * 16
                    nc, cb = accs[2 * t], accs[2 * t + 1]
                    rv = ((ys[t] != zero) | (xs[t] != zero) |
                          (y2s[t] != zero) | (x2s[t] != zero))
                    nc = jnp.where(rv, nc, neg1)
                    cb = jnp.where(rv, cb, neg1)
                    pos = jnp.where((nc >= half) & rv, one, zero)
                    neg = jnp.where((nc < half) & (cb < milli) & rv, one, zero)
                    outb_v[0, pl.ds(o, 16)] = nc
                    outb_v[1, pl.ds(o, 16)] = cb
                    outb_v[2, pl.ds(o, 16)] = pos
                    outb_v[3, pl.ds(o, 16)] = neg
                return 0

            lax.fori_loop(0, SLICE // (16 * K), group_body, 0)
            pltpu.sync_copy(outb_v, out_hbm.at[b, :, pl.ds(base, SLICE)])

    return k(rois_tp, gt_ids_b, gt_boxes_b)


def kernel(rois, gt_ids, gt_boxes):
    B, N, _ = rois.shape
    G = gt_boxes.shape[1]
    rois_t = jnp.transpose(rois, (0, 2, 1))
    bs = _B_SC
    parts = []
    if bs > 0:
        NP = -(-N // (16 * _NW)) * (16 * _NW)
        rois_tp = jnp.pad(rois_t[:bs], ((0, 0), (0, 0), (0, NP - N)))
        ids_p = jnp.pad(gt_ids[:bs], ((0, 0), (0, _GP - G)))
        gts_p = jnp.pad(gt_boxes[:bs], ((0, 0), (0, _GP - G), (0, 0)))
        ids_b = jnp.broadcast_to(ids_p[:, :, None], (bs, _GP, 16))
        gts_b = jnp.broadcast_to(
            jnp.transpose(gts_p, (0, 2, 1))[:, :, :, None], (bs, 4, _GP, 16))
        parts.append(_sc_detection(rois_tp, ids_b, gts_b)[:, :, :N])
    if bs < B:
        parts.append(_tc_detection(rois_t[bs:], gt_ids[bs:], gt_boxes[bs:]))
    if len(parts) == 1:
        return parts[0]
    return jnp.concatenate(parts, axis=0)


# speedup vs baseline: 3.6183x; 3.0641x over previous
import functools

import jax
import jax.numpy as jnp
from jax import lax
from jax.experimental import pallas as pl
from jax.experimental.pallas import tpu as pltpu
from jax.experimental.pallas import tpu_sc as plsc

_NR = 8
_UNROLL = 2
_B_SC = 1
_NW = 32
_GP = 104


def _tc_detection_kernel(rois_ref, ids_ref, gt_ref, out_ref):
    r = rois_ref[0]
    y1 = r[0]
    x1 = r[1]
    y2 = r[2]
    x2 = r[3]
    a1 = (y2 - y1) * (x2 - x1)
    G = gt_ref.shape[1]

    def gbody(g, carry):
        nc, cb = carry
        gy1 = gt_ref[0, g, 0]
        gx1 = gt_ref[0, g, 1]
        gy2 = gt_ref[0, g, 2]
        gx2 = gt_ref[0, g, 3]
        gid = ids_ref[0, g, 0]
        valid = ((jnp.abs(gy1) > 0) | (jnp.abs(gx1) > 0) |
                 (jnp.abs(gy2) > 0) | (jnp.abs(gx2) > 0))
        neg1 = jnp.float32(-1.0)
        is_nc = valid & (gid > 0)
        is_c = valid & (gid < 0)

        a2 = (gy2 - gy1) * (gx2 - gx1)
        iy1 = jnp.maximum(y1, gy1)
        ix1 = jnp.maximum(x1, gx1)
        iy2 = jnp.minimum(y2, gy2)
        ix2 = jnp.minimum(x2, gx2)
        inter = jnp.maximum(iy2 - iy1, 0.0) * jnp.maximum(ix2 - ix1, 0.0)
        union = a1 + a2 - inter
        iou = inter / jnp.maximum(union, 1e-8)
        nc = jnp.maximum(nc, jnp.where(is_nc, iou, neg1))
        cb = jnp.maximum(cb, jnp.where(is_c, iou, neg1))
        return nc, cb

    init = jnp.full_like(a1, -1.0)
    nc_max, c_max = jax.lax.fori_loop(0, G, gbody, (init, init),
                                      unroll=_UNROLL)

    roi_valid = ((jnp.abs(y1) > 0) | (jnp.abs(x1) > 0) |
                 (jnp.abs(y2) > 0) | (jnp.abs(x2) > 0))
    neg_one = jnp.float32(-1.0)
    nc_max = jnp.where(roi_valid, nc_max, neg_one)
    c_max = jnp.where(roi_valid, c_max, neg_one)
    pos = ((nc_max >= 0.5) & roi_valid).astype(jnp.float32)
    neg = ((nc_max < 0.5) & (c_max < 0.001) & roi_valid).astype(jnp.float32)
    out_ref[0, 0] = nc_max
    out_ref[0, 1] = c_max
    out_ref[0, 2] = pos
    out_ref[0, 3] = neg


def _tc_detection(rois_t, gt_ids, gt_boxes):
    B, _, N = rois_t.shape
    G = gt_boxes.shape[1]
    NL = N // _NR
    rb = _NR // 8
    rois_r = rois_t.reshape(B, 4, _NR, NL)
    out = pl.pallas_call(
        _tc_detection_kernel,
        grid=(B, rb),
        in_specs=[
            pl.BlockSpec((1, 4, 8, NL), lambda b, r: (b, 0, r, 0)),
            pl.BlockSpec((1, G, 1), lambda b, r: (b, 0, 0),
                         memory_space=pltpu.SMEM),
            pl.BlockSpec((1, G, 4), lambda b, r: (b, 0, 0),
                         memory_space=pltpu.SMEM),
        ],
        out_specs=pl.BlockSpec((1, 4, 8, NL), lambda b, r: (b, 0, r, 0)),
        out_shape=jax.ShapeDtypeStruct((B, 4, _NR, NL), jnp.float32),
        compiler_params=pltpu.CompilerParams(
            dimension_semantics=("parallel", "parallel"),
        ),
    )(rois_r, gt_ids.reshape(B, G, 1), gt_boxes)
    return out.reshape(B, 4, N)


def _sc_detection(rois_tp, gt_ids_b, gt_boxes_b):
    Bs, _, NP = rois_tp.shape
    GP = gt_ids_b.shape[1]
    SLICE = NP // _NW
    mesh = plsc.VectorSubcoreMesh(core_axis_name="c", subcore_axis_name="s")

    @functools.partial(
        pl.kernel, mesh=mesh,
        out_type=jax.ShapeDtypeStruct((Bs, 4, NP), jnp.float32),
        scratch_types=[
            pltpu.VMEM((4, SLICE), jnp.float32),
            pltpu.VMEM((4, SLICE), jnp.float32),
            pltpu.VMEM((4, GP, 16), jnp.float32),
            pltpu.VMEM((GP, 16), jnp.int32),
            pltpu.VMEM((GP, 16), jnp.float32),
            pltpu.VMEM((GP, 16), jnp.float32),
            pltpu.VMEM((GP, 16), jnp.float32),
        ],
    )
    def k(rois_hbm, ids_hbm, gt_hbm, out_hbm, coords_v, outb_v, gt_v, ids_v,
          a2_v, cnc_v, cc_v):
        wid = lax.axis_index("s") * 2 + lax.axis_index("c")
        base = wid * SLICE
        neg1 = jnp.full((16,), -1.0, jnp.float32)
        zero = jnp.zeros((16,), jnp.float32)
        izero = jnp.zeros((16,), jnp.int32)
        eps = jnp.full((16,), 1e-8, jnp.float32)
        half = jnp.full((16,), 0.5, jnp.float32)
        milli = jnp.full((16,), 0.001, jnp.float32)
        one = jnp.full((16,), 1.0, jnp.float32)
        two = jnp.full((16,), 2.0, jnp.float32)
        K = 4
        for b in range(Bs):
            pltpu.sync_copy(gt_hbm.at[b], gt_v)
            pltpu.sync_copy(ids_hbm.at[b], ids_v)
            pltpu.sync_copy(rois_hbm.at[b, :, pl.ds(base, SLICE)], coords_v)

            def prep_body(g, _):
                gy1 = gt_v[0, g]
                gx1 = gt_v[1, g]
                gy2 = gt_v[2, g]
                gx2 = gt_v[3, g]
                gid = ids_v[g]
                valid = ((gy1 != zero) | (gx1 != zero) |
                         (gy2 != zero) | (gx2 != zero))
                a2_v[g] = (gy2 - gy1) * (gx2 - gx1)
                cnc_v[g] = jnp.where(valid & (gid > izero), two, neg1)
                cc_v[g] = jnp.where(valid & (gid < izero), two, neg1)
                return 0

            lax.fori_loop(0, GP, prep_body, 0)

            def group_body(j, _):
                off = j * 16 * K
                ys, xs, y2s, x2s, a1s = [], [], [], [], []
                for t in range(K):
                    o = off + t * 16
                    y1 = coords_v[0, pl.ds(o, 16)]
                    x1 = coords_v[1, pl.ds(o, 16)]
                    y2 = coords_v[2, pl.ds(o, 16)]
                    x2 = coords_v[3, pl.ds(o, 16)]
                    ys.append(y1)
                    xs.append(x1)
                    y2s.append(y2)
                    x2s.append(x2)
                    a1s.append((y2 - y1) * (x2 - x1))
                init = jnp.full((16,), -1.0, jnp.float32)

                def gt_body(g, carry):
                    accs = list(carry)
                    gy1 = gt_v[0, g]
                    gx1 = gt_v[1, g]
                    gy2 = gt_v[2, g]
                    gx2 = gt_v[3, g]
                    a2 = a2_v[g]
                    capnc = cnc_v[g]
                    capc = cc_v[g]
                    for t in range(K):
                        iy1 = jnp.maximum(ys[t], gy1)
                        ix1 = jnp.maximum(xs[t], gx1)
                        iy2 = jnp.minimum(y2s[t], gy2)
                        ix2 = jnp.minimum(x2s[t], gx2)
                        inter = (jnp.maximum(iy2 - iy1, zero) *
                                 jnp.maximum(ix2 - ix1, zero))
                        union = a1s[t] + a2 - inter
                        iou = inter / jnp.maximum(union, eps)
                        accs[2 * t] = jnp.maximum(
                            accs[2 * t], jnp.minimum(iou, capnc))
                        accs[2 * t + 1] = jnp.maximum(
                            accs[2 * t + 1], jnp.minimum(iou, capc))
                    return tuple(accs)

                accs = lax.fori_loop(0, GP, gt_body, (init,) * (2 * K),
                                     unroll=2)
                for t in range(K):
                    o = off + t * 16
                    nc, cb = accs[2 * t], accs[2 * t + 1]
                    rv = ((ys[t] != zero) | (xs[t] != zero) |
                          (y2s[t] != zero) | (x2s[t] != zero))
                    nc = jnp.where(rv, nc, neg1)
                    cb = jnp.where(rv, cb, neg1)
                    pos = jnp.where((nc >= half) & rv, one, zero)
                    neg = jnp.where((nc < half) & (cb < milli) & rv, one, zero)
                    outb_v[0, pl.ds(o, 16)] = nc
                    outb_v[1, pl.ds(o, 16)] = cb
                    outb_v[2, pl.ds(o, 16)] = pos
                    outb_v[3, pl.ds(o, 16)] = neg
                return 0

            lax.fori_loop(0, SLICE // (16 * K), group_body, 0)
            pltpu.sync_copy(outb_v, out_hbm.at[b, :, pl.ds(base, SLICE)])

    return k(rois_tp, gt_ids_b, gt_boxes_b)


def kernel(rois, gt_ids, gt_boxes):
    B, N, _ = rois.shape
    G = gt_boxes.shape[1]
    rois_t = jnp.transpose(rois, (0, 2, 1))
    bs = _B_SC
    parts = []
    if bs > 0:
        NP = -(-N // (16 * _NW)) * (16 * _NW)
        rois_tp = jnp.pad(rois_t[:bs], ((0, 0), (0, 0), (0, NP - N)))
        ids_p = jnp.pad(gt_ids[:bs], ((0, 0), (0, _GP - G)))
        gts_p = jnp.pad(gt_boxes[:bs], ((0, 0), (0, _GP - G), (0, 0)))
        ids_b = jnp.broadcast_to(ids_p[:, :, None], (bs, _GP, 16))
        gts_b = jnp.broadcast_to(
            jnp.transpose(gts_p, (0, 2, 1))[:, :, :, None], (bs, 4, _GP, 16))
        parts.append(_sc_detection(rois_tp, ids_b, gts_b)[:, :, :N])
    if bs < B:
        parts.append(_tc_detection(rois_t[bs:], gt_ids[bs:], gt_boxes[bs:]))
    if len(parts) == 1:
        return parts[0]
    return jnp.concatenate(parts, axis=0)
